# baseline (speedup 1.0000x reference)
import functools

import jax
import jax.numpy as jnp
from jax import lax
from jax.experimental import pallas as pl
from jax.experimental.pallas import tpu as pltpu
from jax.experimental.pallas import tpu_sc as plsc

_N = 10000
_E = 160000
_W = 256
_G = 8
_GS = _W // _G
_NBOND = 33
_EPS = 1e-5

_BN = 1000
_BE = 640
_PREC = lax.Precision.DEFAULT

_INTERPRET = False
_USE_SC = True

_GWIN = 112
_PERW = 5008
_GNFULL = 44
_GPAIRS = _GNFULL // 2
_GTAIL = _PERW - _GNFULL * _GWIN
_GLASTBASE = _E - _PERW

_SWIN = 64
_PERS = _E // 16
_SNFULL = _PERS // _SWIN
_SQUADS = (_SNFULL - 4) // 4
_STAIL = _PERS - _SNFULL * _SWIN
_HW = _W // 2
_NPS = 624
_NREM = _N - 16 * _NPS


def _cdiv(a, b):
    return (a + b - 1) // b


def _dot(a, b):
    return jnp.dot(a, b, preferred_element_type=jnp.float32, precision=_PREC)


def _gn(xx, gmat, gmatT):
    s1 = _dot(xx, gmat) * (1.0 / _GS)
    s2 = _dot(xx * xx, gmat) * (1.0 / _GS)
    mu = _dot(s1, gmatT)
    ex2 = _dot(s2, gmatT)
    var = ex2 - mu * mu
    return (xx - mu) * lax.rsqrt(var + _EPS)


def _node_mm_body(x_ref, wa_ref, wb_ref, oa_ref, ob_ref):
    x = x_ref[...]
    oa_ref[...] = _dot(x, wa_ref[...]).astype(jnp.bfloat16).reshape(
        x.shape[0], 2, _W // 2)
    ob_ref[...] = _dot(x, wb_ref[...]).astype(jnp.bfloat16).reshape(
        x.shape[0], 2, _W // 2)


def _node_mm(x, wa, wb):
    return pl.pallas_call(
        _node_mm_body,
        grid=(_cdiv(_N, _BN),),
        in_specs=[
            pl.BlockSpec((_BN, _W), lambda i: (i, 0)),
            pl.BlockSpec((_W, _W), lambda i: (0, 0)),
            pl.BlockSpec((_W, _W), lambda i: (0, 0)),
        ],
        out_specs=[pl.BlockSpec((_BN, 2, _W // 2), lambda i: (i, 0, 0))] * 2,
        out_shape=[jax.ShapeDtypeStruct((_N, 2, _W // 2), jnp.bfloat16)] * 2,
        interpret=_INTERPRET,
    )(x, wa, wb)


def _edge_ffn_body(xx_ref, attr_ref, table_ref, gmat_ref, gmatT_ref,
                   gw_ref, vw_ref, o_ref):
    xx = xx_ref[...].reshape(-1, _W).astype(jnp.float32)
    xn = _gn(xx, gmat_ref[...], gmatT_ref[...])
    a = attr_ref[...]
    iot = lax.broadcasted_iota(jnp.int32, (1, _NBOND), 1)
    cnts = jnp.zeros((a.shape[0], _NBOND), jnp.float32)
    for j in range(3):
        cnts = cnts + (a[:, j:j + 1] == iot).astype(jnp.float32)
    nz = (iot != 0).astype(jnp.float32)
    cnts = cnts * nz
    denom = jnp.maximum(jnp.sum(cnts, axis=1, keepdims=True), 1.0)
    gb = _dot(cnts, table_ref[...]) / denom
    g = jnp.maximum(_dot(xn + gb, gw_ref[...]), 0.0)
    v = _dot(xn, vw_ref[...])
    o_ref[...] = g * v


def _edge_ffn(xx, attr, table, gmat, gmatT, gwbd, vwbd):
    full = lambda r, c: pl.BlockSpec((r, c), lambda i: (0, 0))
    return pl.pallas_call(
        _edge_ffn_body,
        grid=(_cdiv(_E, _BE),),
        in_specs=[
            pl.BlockSpec((_BE, 2, _W // 2), lambda i: (i, 0, 0)),
            pl.BlockSpec((_BE, 3), lambda i: (i, 0)),
            full(_NBOND, _W),
            full(_W, _G),
            full(_G, _W),
            full(_W, _W),
            full(_W, _W),
        ],
        out_specs=pl.BlockSpec((_BE, _W), lambda i: (i, 0)),
        out_shape=jax.ShapeDtypeStruct((_E, _W), jnp.float32),
        interpret=_INTERPRET,
    )(xx, attr, table, gmat, gmatT, gwbd, vwbd)


def _post_body(z_ref, w_ref, deg_ref, dp_ref, *rest):
    agg = _dot(z_ref[...], w_ref[...])
    scale = jnp.exp(dp_ref[...] * jnp.log(deg_ref[...]))
    hop = agg * scale
    return hop, rest


def _post_first_body(z_ref, w_ref, deg_ref, dp_ref, oh_ref):
    hop, _ = _post_body(z_ref, w_ref, deg_ref, dp_ref)
    oh_ref[...] = hop


def _post_mid_body(z_ref, w_ref, deg_ref, dp_ref, xo_ref, oh_ref, oo_ref):
    hop, _ = _post_body(z_ref, w_ref, deg_ref, dp_ref)
    oh_ref[...] = hop
    oo_ref[...] = xo_ref[...] + hop


def _post_last_body(z_ref, w_ref, deg_ref, dp_ref, xo_ref, res_ref, oo_ref):
    hop, _ = _post_body(z_ref, w_ref, deg_ref, dp_ref)
    oo_ref[...] = xo_ref[...] + hop + res_ref[...]


def _post(zagg, pwT, deg, dp, xout, res):
    row = pl.BlockSpec((_BN, _W), lambda i: (i, 0))
    specs = [
        row,
        pl.BlockSpec((_W, _W), lambda i: (0, 0)),
        pl.BlockSpec((_BN, 1), lambda i: (i, 0)),
        pl.BlockSpec((1, _W), lambda i: (0, 0)),
    ]
    args = [zagg, pwT, deg, dp]
    shp = jax.ShapeDtypeStruct((_N, _W), jnp.float32)
    if xout is None:
        out = pl.pallas_call(
            _post_first_body, grid=(_cdiv(_N, _BN),),
            in_specs=specs, out_specs=row, out_shape=shp,
            interpret=_INTERPRET)(*args)
        return out, out
    if res is None:
        hop, xo = pl.pallas_call(
            _post_mid_body, grid=(_cdiv(_N, _BN),),
            in_specs=specs + [row], out_specs=[row, row], out_shape=[shp, shp],
            interpret=_INTERPRET)(*args, xout)
        return hop, xo
    xo = pl.pallas_call(
        _post_last_body, grid=(_cdiv(_N, _BN),),
        in_specs=specs + [row, row], out_specs=row, out_shape=shp,
        interpret=_INTERPRET)(*args, xout, res)
    return None, xo


def _mix_body(x_ref, res_ref, preT_ref, gw_ref, vw_ref, postT_ref,
              sp_ref, spo_ref, gmat_ref, gmatT_ref, o_ref):
    xx = jnp.exp(sp_ref[...]) * x_ref[...] + res_ref[...]
    xn = _gn(_dot(xx, preT_ref[...]), gmat_ref[...], gmatT_ref[...])
    g = jnp.maximum(_dot(xn, gw_ref[...]), 0.0)
    v = _dot(xn, vw_ref[...])
    o_ref[...] = jnp.exp(spo_ref[...]) * xx + _dot(g * v, postT_ref[...])


def _mix(x, res, preT, gwbd, vwbd, postT, sp, spo, gmat, gmatT):
    row = pl.BlockSpec((_BN, _W), lambda i: (i, 0))
    full = lambda r, c: pl.BlockSpec((r, c), lambda i: (0, 0))
    return pl.pallas_call(
        _mix_body,
        grid=(_cdiv(_N, _BN),),
        in_specs=[row, row, full(_W, _W), full(_W, _W), full(_W, _W),
                  full(_W, _W), full(1, _W), full(1, _W),
                  full(_W, _G), full(_G, _W)],
        out_specs=row,
        out_shape=jax.ShapeDtypeStruct((_N, _W), jnp.float32),
        interpret=_INTERPRET,
    )(x, res, preT, gwbd, vwbd, postT, sp, spo, gmat, gmatT)


def _sc_gather(xs, xt, src, dst):
    if not _USE_SC:
        return jnp.take(xs, src, axis=0) + jnp.take(xt, dst, axis=0)
    mesh = plsc.VectorSubcoreMesh(core_axis_name="c", subcore_axis_name="s")

    @functools.partial(
        pl.kernel, mesh=mesh,
        out_type=jax.ShapeDtypeStruct((_E, 2, _W // 2), jnp.bfloat16),
        compiler_params=pltpu.CompilerParams(use_tc_tiling_on_sc=False),
        scratch_types=[
            pltpu.VMEM((_GWIN,), jnp.int32),
            pltpu.VMEM((_GWIN,), jnp.int32),
            pltpu.VMEM((_GWIN,), jnp.int32),
            pltpu.VMEM((_GWIN,), jnp.int32),
            pltpu.VMEM((_GWIN, 2, _W // 2), jnp.bfloat16),
            pltpu.VMEM((_GWIN, 2, _W // 2), jnp.bfloat16),
            pltpu.VMEM((_GWIN, 2, _W // 2), jnp.bfloat16),
            pltpu.VMEM((_GWIN, 2, _W // 2), jnp.bfloat16),
        ] + [pltpu.SemaphoreType.DMA] * 8)
    def k(xs_hbm, xt_hbm, src_hbm, dst_hbm, os_hbm,
          si0, si1, di0, di1, bs0, bs1, bt0, bt1,
          six0, six1, sga0, sga1, sgb0, sgb1, swa0, swa1):
        si = (si0, si1)
        di = (di0, di1)
        bs = (bs0, bs1)
        bt = (bt0, bt1)
        six = (six0, six1)
        sga = (sga0, sga1)
        sgb = (sgb0, sgb1)
        swa = (swa0, swa1)
        wid = lax.axis_index("s") * 2 + lax.axis_index("c")
        base = jnp.where(wid == 31, _GLASTBASE, wid * _PERW)

        def issue_idx(w, p):
            pltpu.async_copy(src_hbm.at[pl.ds(base + w * _GWIN, _GWIN)],
                             si[p], six[p])
            pltpu.async_copy(dst_hbm.at[pl.ds(base + w * _GWIN, _GWIN)],
                             di[p], six[p])

        def wait_idx(p):
            pltpu.make_async_copy(src_hbm.at[pl.ds(0, _GWIN)],
                                  si[p], six[p]).wait()
            pltpu.make_async_copy(dst_hbm.at[pl.ds(0, _GWIN)],
                                  di[p], six[p]).wait()

        def issue_gather(p):
            pltpu.async_copy(xs_hbm.at[si[p]], bs[p], sga[p])
            pltpu.async_copy(xt_hbm.at[di[p]], bt[p], sgb[p])

        def wait_gather(p):
            pltpu.make_async_copy(xs_hbm.at[si[p]], bs[p], sga[p]).wait()
            pltpu.make_async_copy(xt_hbm.at[di[p]], bt[p], sgb[p]).wait()

        def add_rows(p, nrows):
            a = bs[p]
            b = bt[p]

            @pl.loop(0, nrows)
            def _(r):
                for j in range(_W // 2 // 16):
                    sl = (r, pl.ds(0, 2), pl.ds(j * 16, 16))
                    a[sl] = a[sl] + b[sl]

        def issue_wb(w, p):
            pltpu.async_copy(bs[p], os_hbm.at[pl.ds(base + w * _GWIN, _GWIN)],
                             swa[p])

        def wait_wb(p):
            pltpu.make_async_copy(bs[p], os_hbm.at[pl.ds(0, _GWIN)],
                                  swa[p]).wait()

        issue_idx(0, 0)
        issue_idx(1, 1)
        wait_idx(0)
        issue_gather(0)
        wait_idx(1)
        issue_gather(1)
        wait_gather(0)
        add_rows(0, _GWIN)
        issue_wb(0, 0)
        issue_idx(2, 0)

        @pl.loop(1, _GPAIRS)
        def _(j):
            wa = 2 * j
            wait_wb(0)
            wait_idx(0)
            issue_gather(0)
            wait_gather(1)
            add_rows(1, _GWIN)
            issue_wb(wa - 1, 1)
            issue_idx(wa + 1, 1)
            wait_wb(1)
            wait_idx(1)
            issue_gather(1)
            wait_gather(0)
            add_rows(0, _GWIN)
            issue_wb(wa, 0)

            @pl.when(j < _GPAIRS - 1)
            def _():
                issue_idx(wa + 2, 0)

        wait_gather(1)
        add_rows(1, _GWIN)
        issue_wb(_GNFULL - 1, 1)

        toff = base + _GNFULL * _GWIN
        wait_wb(0)
        h1 = pltpu.async_copy(src_hbm.at[pl.ds(toff, _GTAIL)],
                              si0.at[pl.ds(0, _GTAIL)], six0)
        h2 = pltpu.async_copy(dst_hbm.at[pl.ds(toff, _GTAIL)],
                              di0.at[pl.ds(0, _GTAIL)], six0)
        h1.wait()
        h2.wait()
        g1 = pltpu.async_copy(xs_hbm.at[si0.at[pl.ds(0, _GTAIL)]],
                              bs0.at[pl.ds(0, _GTAIL)], sga0)
        g2 = pltpu.async_copy(xt_hbm.at[di0.at[pl.ds(0, _GTAIL)]],
                              bt0.at[pl.ds(0, _GTAIL)], sgb0)
        g1.wait()
        g2.wait()
        add_rows(0, _GTAIL)
        o1 = pltpu.async_copy(bs0.at[pl.ds(0, _GTAIL)],
                              os_hbm.at[pl.ds(toff, _GTAIL)], swa0)
        o1.wait()
        wait_wb(1)

    return k(xs, xt, src, dst)


def _sc_scatter(z, dst):
    if not _USE_SC:
        return jnp.zeros((_N, _W), z.dtype).at[dst].add(z)
    mesh = plsc.VectorSubcoreMesh(core_axis_name="c", subcore_axis_name="s")

    @functools.partial(
        pl.kernel, mesh=mesh,
        out_type=jax.ShapeDtypeStruct((_N, _W), jnp.float32),
        scratch_types=[
            pltpu.VMEM((_SWIN,), jnp.int32),
            pltpu.VMEM((_SWIN,), jnp.int32),
            pltpu.VMEM((_SWIN,), jnp.int32),
            pltpu.VMEM((_SWIN,), jnp.int32),
            pltpu.VMEM((_STAIL,), jnp.int32),
            pltpu.VMEM((_SWIN, _HW), jnp.float32),
            pltpu.VMEM((_SWIN, _HW), jnp.float32),
            pltpu.VMEM((_SWIN, _HW), jnp.float32),
            pltpu.VMEM((_SWIN, _HW), jnp.float32),
            pltpu.VMEM((_STAIL, _HW), jnp.float32),
            pltpu.VMEM_SHARED((_N, _HW), jnp.float32),
        ] + [pltpu.SemaphoreType.DMA] * 8)
    def k(z_hbm, dst_hbm, o_hbm, idx0, idx1, idx2, idx3, idxt,
          zb0, zb1, zb2, zb3, zbt, acc,
          sl0, sl1, sl2, sl3, ss0, ss1, ss2, ss3):
        idx = (idx0, idx1, idx2, idx3)
        zb = (zb0, zb1, zb2, zb3)
        sl = (sl0, sl1, sl2, sl3)
        ss = (ss0, ss1, ss2, ss3)
        c = lax.axis_index("c")
        s = lax.axis_index("s")

        @pl.loop(0, _SWIN)
        def _(r):
            for j in range(_HW // 16):
                zb0[r, pl.ds(j * 16, 16)] = jnp.zeros((16,), jnp.float32)

        rbase = s * _NPS
        zchunks = [(o, _SWIN) for o in range(0, _NPS - _SWIN + 1, _SWIN)]
        if _NPS % _SWIN:
            zchunks.append((_NPS - _NPS % _SWIN, _NPS % _SWIN))
        for (o, n) in zchunks:
            pltpu.sync_copy(zb0.at[pl.ds(0, n)], acc.at[pl.ds(rbase + o, n)])

        @pl.when(s == 15)
        def _():
            pltpu.sync_copy(zb0.at[pl.ds(0, _NREM)],
                            acc.at[pl.ds(16 * _NPS, _NREM)])
        plsc.subcore_barrier()

        ebase = s * _PERS

        def issue_loads(w, p):
            off = ebase + w * _SWIN
            pltpu.async_copy(dst_hbm.at[pl.ds(off, _SWIN)], idx[p], sl[p])
            pltpu.async_copy(z_hbm.at[pl.ds(off, _SWIN), pl.ds(c * _HW, _HW)],
                             zb[p], sl[p])

        def wait_loads(p):
            pltpu.make_async_copy(dst_hbm.at[pl.ds(0, _SWIN)],
                                  idx[p], sl[p]).wait()
            pltpu.make_async_copy(z_hbm.at[pl.ds(0, _SWIN), pl.ds(0, _HW)],
                                  zb[p], sl[p]).wait()

        def issue_scatter(p):
            pltpu.async_copy(zb[p], acc.at[idx[p]], ss[p], add=True)

        def wait_scatter(p):
            pltpu.make_async_copy(zb[p], acc.at[idx[p]], ss[p]).wait()

        issue_loads(0, 0)
        issue_loads(1, 1)
        wait_loads(0)
        issue_scatter(0)
        issue_loads(2, 2)
        wait_loads(1)
        issue_scatter(1)
        issue_loads(3, 3)

        @pl.loop(0, _SQUADS)
        def _(j):
            w0 = 2 + 4 * j
            for r in range(4):
                p = (2 + r) % 4
                q = (p + 2) % 4
                wait_loads(p)
                issue_scatter(p)
                wait_scatter(q)

                @pl.when(w0 + r + 2 < _SNFULL)
                def _():
                    issue_loads(w0 + r + 2, q)

        wait_loads(2)
        issue_scatter(2)
        wait_scatter(0)
        wait_loads(3)
        issue_scatter(3)
        wait_scatter(1)
        wait_scatter(2)
        wait_scatter(3)

        toff = ebase + _SNFULL * _SWIN
        pltpu.sync_copy(dst_hbm.at[pl.ds(toff, _STAIL)], idxt)
        pltpu.sync_copy(z_hbm.at[pl.ds(toff, _STAIL), pl.ds(c * _HW, _HW)], zbt)
        pltpu.sync_copy(zbt, acc.at[idxt], add=True)
        plsc.subcore_barrier()

        pltpu.sync_copy(acc.at[pl.ds(rbase, _NPS)],
                        o_hbm.at[pl.ds(rbase, _NPS), pl.ds(c * _HW, _HW)])

        @pl.when(s == 15)
        def _():
            pltpu.sync_copy(
                acc.at[pl.ds(16 * _NPS, _NREM)],
                o_hbm.at[pl.ds(16 * _NPS, _NREM), pl.ds(c * _HW, _HW)])

    return k(z, dst)


def _block_diag(w):
    lead = w.shape[:-3]
    out = jnp.zeros(lead + (_G, _GS, _G, _GS), w.dtype)
    wt = jnp.swapaxes(w, -1, -2)
    for g in range(_G):
        out = out.at[..., g, :, g, :].set(wt[..., g, :, :])
    return out.reshape(lead + (_W, _W))


def kernel(x, x_res, edge_index_0, edge_index_1, edge_index_2,
           edge_attr_0, edge_attr_1, edge_attr_2,
           node_deg_0, node_deg_1, node_deg_2,
           src_w, tgt_w, bond_tables, gate_w, value_w, post_w, deg_p,
           mix_pre_w, mix_gate_w, mix_value_w, mix_post_w,
           mix_sca_pre, mix_sca_post):
    eis = [edge_index_0, edge_index_1, edge_index_2]
    eas = [edge_attr_0, edge_attr_1, edge_attr_2]
    degs = [node_deg_0, node_deg_1, node_deg_2]
    srcs = [e[0] for e in eis]
    dsts = [e[1] for e in eis]
    deg2 = [d[:, None] for d in degs]

    swT = jnp.swapaxes(src_w, -1, -2)
    twT = jnp.swapaxes(tgt_w, -1, -2)
    pwT = jnp.swapaxes(post_w, -1, -2)
    gwbd = _block_diag(gate_w)
    vwbd = _block_diag(value_w)
    mix_gwbd = _block_diag(mix_gate_w)
    mix_vwbd = _block_diag(mix_value_w)
    mix_preT = mix_pre_w.T
    mix_postT = mix_post_w.T
    sp2 = mix_sca_pre[None, :]
    spo2 = mix_sca_post[None, :]

    gmat = jnp.kron(jnp.eye(_G, dtype=jnp.float32),
                    jnp.ones((_GS, 1), jnp.float32))
    gmatT = gmat.T

    x_hop = x
    x_kernel = x
    x_out = None
    for k in range(2):
        for h in range(3):
            i = k * 3 + h
            xs, xt = _node_mm(x_hop, swT[i], twT[i])
            xx = _sc_gather(xs, xt, srcs[h], dsts[h])
            z = _edge_ffn(xx, eas[h], bond_tables[i], gmat, gmatT,
                          gwbd[i], vwbd[i])
            zagg = _sc_scatter(z, dsts[h])
            x_hop, x_out = _post(zagg, pwT[i], deg2[h], deg_p[i][None, :],
                                 x_out, x_res if i == 5 else None)
        if k == 0:
            x_kernel = _mix(x_kernel, x_out, mix_preT, mix_gwbd, mix_vwbd,
                            mix_postT, sp2, spo2, gmat, gmatT)
            x_hop = x_kernel
            x_out = None
    return x_out


# speedup vs baseline: 1.5071x; 1.5071x over previous
import functools

import jax
import jax.numpy as jnp
from jax import lax
from jax.experimental import pallas as pl
from jax.experimental.pallas import tpu as pltpu
from jax.experimental.pallas import tpu_sc as plsc

_N = 10000
_E = 160000
_W = 256
_G = 8
_GS = _W // _G
_NBOND = 33
_EPS = 1e-5

_BN = 1000
_BE = 640
_PREC = lax.Precision.DEFAULT

_GWIN = 112
_PERW = 5008
_GNFULL = 44
_GPAIRS = _GNFULL // 2
_GTAIL = _PERW - _GNFULL * _GWIN
_GLASTBASE = _E - _PERW

_SWIN = 64
_PERS = _E // 16
_SNFULL = _PERS // _SWIN
_SQUADS = (_SNFULL - 4) // 4
_STAIL = _PERS - _SNFULL * _SWIN
_HW = _W // 2
_NPS = 624
_NREM = _N - 16 * _NPS


def _cdiv(a, b):
    return (a + b - 1) // b


def _dot(a, b):
    return jnp.dot(a, b, preferred_element_type=jnp.float32, precision=_PREC)


def _gn(xx, gmat, gmatT):
    s1 = _dot(xx, gmat) * (1.0 / _GS)
    s2 = _dot(xx * xx, gmat) * (1.0 / _GS)
    mu = _dot(s1, gmatT)
    ex2 = _dot(s2, gmatT)
    var = ex2 - mu * mu
    return (xx - mu) * lax.rsqrt(var + _EPS)


def _node_mm_body(x_ref, wa_ref, wb_ref, oa_ref, ob_ref):
    x = x_ref[...]
    oa_ref[...] = _dot(x, wa_ref[...])
    ob_ref[...] = _dot(x, wb_ref[...])


def _node_mm(x, wa, wb):
    return pl.pallas_call(
        _node_mm_body,
        grid=(_cdiv(_N, _BN),),
        in_specs=[
            pl.BlockSpec((_BN, _W), lambda i: (i, 0)),
            pl.BlockSpec((_W, _W), lambda i: (0, 0)),
            pl.BlockSpec((_W, _W), lambda i: (0, 0)),
        ],
        out_specs=[pl.BlockSpec((_BN, _W), lambda i: (i, 0))] * 2,
        out_shape=[jax.ShapeDtypeStruct((_N, _W), jnp.float32)] * 2,
    )(x, wa, wb)


def _edge_ffn_body(xx_ref, attr_ref, table_ref, gmat_ref, gmatT_ref,
                   gw_ref, vw_ref, o_ref):
    xx = xx_ref[...]
    xn = _gn(xx, gmat_ref[...], gmatT_ref[...])
    a = attr_ref[...]
    iot = lax.broadcasted_iota(jnp.int32, (1, _NBOND), 1)
    cnts = jnp.zeros((a.shape[0], _NBOND), jnp.float32)
    for j in range(3):
        cnts = cnts + (a[:, j:j + 1] == iot).astype(jnp.float32)
    nz = (iot != 0).astype(jnp.float32)
    cnts = cnts * nz
    denom = jnp.maximum(jnp.sum(cnts, axis=1, keepdims=True), 1.0)
    gb = _dot(cnts, table_ref[...]) / denom
    g = jnp.maximum(_dot(xn + gb, gw_ref[...]), 0.0)
    v = _dot(xn, vw_ref[...])
    o_ref[...] = g * v


def _edge_ffn(xx, attr, table, gmat, gmatT, gwbd, vwbd):
    full = lambda r, c: pl.BlockSpec((r, c), lambda i: (0, 0))
    return pl.pallas_call(
        _edge_ffn_body,
        grid=(_cdiv(_E, _BE),),
        in_specs=[
            pl.BlockSpec((_BE, _W), lambda i: (i, 0)),
            pl.BlockSpec((_BE, 3), lambda i: (i, 0)),
            full(_NBOND, _W),
            full(_W, _G),
            full(_G, _W),
            full(_W, _W),
            full(_W, _W),
        ],
        out_specs=pl.BlockSpec((_BE, _W), lambda i: (i, 0)),
        out_shape=jax.ShapeDtypeStruct((_E, _W), jnp.float32),
    )(xx, attr, table, gmat, gmatT, gwbd, vwbd)


def _post_body(z_ref, w_ref, deg_ref, dp_ref, *rest):
    agg = _dot(z_ref[...], w_ref[...])
    scale = jnp.exp(dp_ref[...] * jnp.log(deg_ref[...]))
    hop = agg * scale
    return hop, rest


def _post_first_body(z_ref, w_ref, deg_ref, dp_ref, oh_ref):
    hop, _ = _post_body(z_ref, w_ref, deg_ref, dp_ref)
    oh_ref[...] = hop


def _post_mid_body(z_ref, w_ref, deg_ref, dp_ref, xo_ref, oh_ref, oo_ref):
    hop, _ = _post_body(z_ref, w_ref, deg_ref, dp_ref)
    oh_ref[...] = hop
    oo_ref[...] = xo_ref[...] + hop


def _post_last_body(z_ref, w_ref, deg_ref, dp_ref, xo_ref, res_ref, oo_ref):
    hop, _ = _post_body(z_ref, w_ref, deg_ref, dp_ref)
    oo_ref[...] = xo_ref[...] + hop + res_ref[...]


def _post(zagg, pwT, deg, dp, xout, res):
    row = pl.BlockSpec((_BN, _W), lambda i: (i, 0))
    specs = [
        row,
        pl.BlockSpec((_W, _W), lambda i: (0, 0)),
        pl.BlockSpec((_BN, 1), lambda i: (i, 0)),
        pl.BlockSpec((1, _W), lambda i: (0, 0)),
    ]
    args = [zagg, pwT, deg, dp]
    shp = jax.ShapeDtypeStruct((_N, _W), jnp.float32)
    if xout is None:
        out = pl.pallas_call(
            _post_first_body, grid=(_cdiv(_N, _BN),),
            in_specs=specs, out_specs=row, out_shape=shp,
            )(*args)
        return out, out
    if res is None:
        hop, xo = pl.pallas_call(
            _post_mid_body, grid=(_cdiv(_N, _BN),),
            in_specs=specs + [row], out_specs=[row, row], out_shape=[shp, shp],
            )(*args, xout)
        return hop, xo
    xo = pl.pallas_call(
        _post_last_body, grid=(_cdiv(_N, _BN),),
        in_specs=specs + [row, row], out_specs=row, out_shape=shp,
        )(*args, xout, res)
    return None, xo


def _mix_body(x_ref, res_ref, preT_ref, gw_ref, vw_ref, postT_ref,
              sp_ref, spo_ref, gmat_ref, gmatT_ref, o_ref):
    xx = jnp.exp(sp_ref[...]) * x_ref[...] + res_ref[...]
    xn = _gn(_dot(xx, preT_ref[...]), gmat_ref[...], gmatT_ref[...])
    g = jnp.maximum(_dot(xn, gw_ref[...]), 0.0)
    v = _dot(xn, vw_ref[...])
    o_ref[...] = jnp.exp(spo_ref[...]) * xx + _dot(g * v, postT_ref[...])


def _mix(x, res, preT, gwbd, vwbd, postT, sp, spo, gmat, gmatT):
    row = pl.BlockSpec((_BN, _W), lambda i: (i, 0))
    full = lambda r, c: pl.BlockSpec((r, c), lambda i: (0, 0))
    return pl.pallas_call(
        _mix_body,
        grid=(_cdiv(_N, _BN),),
        in_specs=[row, row, full(_W, _W), full(_W, _W), full(_W, _W),
                  full(_W, _W), full(1, _W), full(1, _W),
                  full(_W, _G), full(_G, _W)],
        out_specs=row,
        out_shape=jax.ShapeDtypeStruct((_N, _W), jnp.float32),
    )(x, res, preT, gwbd, vwbd, postT, sp, spo, gmat, gmatT)


def _sc_gather(xs, xt, src, dst):
    mesh = plsc.VectorSubcoreMesh(core_axis_name="c", subcore_axis_name="s")

    @functools.partial(
        pl.kernel, mesh=mesh,
        out_type=jax.ShapeDtypeStruct((_E, _W), jnp.float32),
        scratch_types=[
            pltpu.VMEM((_GWIN,), jnp.int32),
            pltpu.VMEM((_GWIN,), jnp.int32),
            pltpu.VMEM((_GWIN,), jnp.int32),
            pltpu.VMEM((_GWIN,), jnp.int32),
            pltpu.VMEM((_GWIN, _W), jnp.float32),
            pltpu.VMEM((_GWIN, _W), jnp.float32),
            pltpu.VMEM((_GWIN, _W), jnp.float32),
            pltpu.VMEM((_GWIN, _W), jnp.float32),
        ] + [pltpu.SemaphoreType.DMA] * 8)
    def k(xs_hbm, xt_hbm, src_hbm, dst_hbm, os_hbm,
          si0, si1, di0, di1, bs0, bs1, bt0, bt1,
          six0, six1, sga0, sga1, sgb0, sgb1, swa0, swa1):
        si = (si0, si1)
        di = (di0, di1)
        bs = (bs0, bs1)
        bt = (bt0, bt1)
        six = (six0, six1)
        sga = (sga0, sga1)
        sgb = (sgb0, sgb1)
        swa = (swa0, swa1)
        wid = lax.axis_index("s") * 2 + lax.axis_index("c")
        base = jnp.where(wid == 31, _GLASTBASE, wid * _PERW)

        def issue_idx(w, p):
            pltpu.async_copy(src_hbm.at[pl.ds(base + w * _GWIN, _GWIN)],
                             si[p], six[p])
            pltpu.async_copy(dst_hbm.at[pl.ds(base + w * _GWIN, _GWIN)],
                             di[p], six[p])

        def wait_idx(p):
            pltpu.make_async_copy(src_hbm.at[pl.ds(0, _GWIN)],
                                  si[p], six[p]).wait()
            pltpu.make_async_copy(dst_hbm.at[pl.ds(0, _GWIN)],
                                  di[p], six[p]).wait()

        def issue_gather(p):
            pltpu.async_copy(xs_hbm.at[si[p]], bs[p], sga[p])
            pltpu.async_copy(xt_hbm.at[di[p]], bt[p], sgb[p])

        def wait_gather(p):
            pltpu.make_async_copy(xs_hbm.at[si[p]], bs[p], sga[p]).wait()
            pltpu.make_async_copy(xt_hbm.at[di[p]], bt[p], sgb[p]).wait()

        def add_rows(p, nrows):
            a = bs[p]
            b = bt[p]

            @pl.loop(0, nrows)
            def _(r):
                for j in range(_W // 16):
                    sl = pl.ds(j * 16, 16)
                    a[r, sl] = a[r, sl] + b[r, sl]

        def issue_wb(w, p):
            pltpu.async_copy(bs[p], os_hbm.at[pl.ds(base + w * _GWIN, _GWIN)],
                             swa[p])

        def wait_wb(p):
            pltpu.make_async_copy(bs[p], os_hbm.at[pl.ds(0, _GWIN)],
                                  swa[p]).wait()

        issue_idx(0, 0)
        issue_idx(1, 1)
        wait_idx(0)
        issue_gather(0)
        wait_idx(1)
        issue_gather(1)
        wait_gather(0)
        add_rows(0, _GWIN)
        issue_wb(0, 0)
        issue_idx(2, 0)

        @pl.loop(1, _GPAIRS)
        def _(j):
            wa = 2 * j
            wait_wb(0)
            wait_idx(0)
            issue_gather(0)
            wait_gather(1)
            add_rows(1, _GWIN)
            issue_wb(wa - 1, 1)
            issue_idx(wa + 1, 1)
            wait_wb(1)
            wait_idx(1)
            issue_gather(1)
            wait_gather(0)
            add_rows(0, _GWIN)
            issue_wb(wa, 0)

            @pl.when(j < _GPAIRS - 1)
            def _():
                issue_idx(wa + 2, 0)

        wait_gather(1)
        add_rows(1, _GWIN)
        issue_wb(_GNFULL - 1, 1)

        toff = base + _GNFULL * _GWIN
        wait_wb(0)
        h1 = pltpu.async_copy(src_hbm.at[pl.ds(toff, _GTAIL)],
                              si0.at[pl.ds(0, _GTAIL)], six0)
        h2 = pltpu.async_copy(dst_hbm.at[pl.ds(toff, _GTAIL)],
                              di0.at[pl.ds(0, _GTAIL)], six0)
        h1.wait()
        h2.wait()
        g1 = pltpu.async_copy(xs_hbm.at[si0.at[pl.ds(0, _GTAIL)]],
                              bs0.at[pl.ds(0, _GTAIL)], sga0)
        g2 = pltpu.async_copy(xt_hbm.at[di0.at[pl.ds(0, _GTAIL)]],
                              bt0.at[pl.ds(0, _GTAIL)], sgb0)
        g1.wait()
        g2.wait()
        add_rows(0, _GTAIL)
        o1 = pltpu.async_copy(bs0.at[pl.ds(0, _GTAIL)],
                              os_hbm.at[pl.ds(toff, _GTAIL)], swa0)
        o1.wait()
        wait_wb(1)

    return k(xs, xt, src, dst)


def _sc_scatter(z, dst):
    mesh = plsc.VectorSubcoreMesh(core_axis_name="c", subcore_axis_name="s")

    @functools.partial(
        pl.kernel, mesh=mesh,
        out_type=jax.ShapeDtypeStruct((_N, _W), jnp.float32),
        scratch_types=[
            pltpu.VMEM((_SWIN,), jnp.int32),
            pltpu.VMEM((_SWIN,), jnp.int32),
            pltpu.VMEM((_SWIN,), jnp.int32),
            pltpu.VMEM((_SWIN,), jnp.int32),
            pltpu.VMEM((_STAIL,), jnp.int32),
            pltpu.VMEM((_SWIN, _HW), jnp.float32),
            pltpu.VMEM((_SWIN, _HW), jnp.float32),
            pltpu.VMEM((_SWIN, _HW), jnp.float32),
            pltpu.VMEM((_SWIN, _HW), jnp.float32),
            pltpu.VMEM((_STAIL, _HW), jnp.float32),
            pltpu.VMEM_SHARED((_N, _HW), jnp.float32),
        ] + [pltpu.SemaphoreType.DMA] * 8)
    def k(z_hbm, dst_hbm, o_hbm, idx0, idx1, idx2, idx3, idxt,
          zb0, zb1, zb2, zb3, zbt, acc,
          sl0, sl1, sl2, sl3, ss0, ss1, ss2, ss3):
        idx = (idx0, idx1, idx2, idx3)
        zb = (zb0, zb1, zb2, zb3)
        sl = (sl0, sl1, sl2, sl3)
        ss = (ss0, ss1, ss2, ss3)
        c = lax.axis_index("c")
        s = lax.axis_index("s")

        @pl.loop(0, _SWIN)
        def _(r):
            for j in range(_HW // 16):
                zb0[r, pl.ds(j * 16, 16)] = jnp.zeros((16,), jnp.float32)

        rbase = s * _NPS
        zchunks = [(o, _SWIN) for o in range(0, _NPS - _SWIN + 1, _SWIN)]
        if _NPS % _SWIN:
            zchunks.append((_NPS - _NPS % _SWIN, _NPS % _SWIN))
        for (o, n) in zchunks:
            pltpu.sync_copy(zb0.at[pl.ds(0, n)], acc.at[pl.ds(rbase + o, n)])

        @pl.when(s == 15)
        def _():
            pltpu.sync_copy(zb0.at[pl.ds(0, _NREM)],
                            acc.at[pl.ds(16 * _NPS, _NREM)])
        plsc.subcore_barrier()

        ebase = s * _PERS

        def issue_loads(w, p):
            off = ebase + w * _SWIN
            pltpu.async_copy(dst_hbm.at[pl.ds(off, _SWIN)], idx[p], sl[p])
            pltpu.async_copy(z_hbm.at[pl.ds(off, _SWIN), pl.ds(c * _HW, _HW)],
                             zb[p], sl[p])

        def wait_loads(p):
            pltpu.make_async_copy(dst_hbm.at[pl.ds(0, _SWIN)],
                                  idx[p], sl[p]).wait()
            pltpu.make_async_copy(z_hbm.at[pl.ds(0, _SWIN), pl.ds(0, _HW)],
                                  zb[p], sl[p]).wait()

        def issue_scatter(p):
            pltpu.async_copy(zb[p], acc.at[idx[p]], ss[p], add=True)

        def wait_scatter(p):
            pltpu.make_async_copy(zb[p], acc.at[idx[p]], ss[p]).wait()

        issue_loads(0, 0)
        issue_loads(1, 1)
        wait_loads(0)
        issue_scatter(0)
        issue_loads(2, 2)
        wait_loads(1)
        issue_scatter(1)
        issue_loads(3, 3)

        @pl.loop(0, _SQUADS)
        def _(j):
            w0 = 2 + 4 * j
            for r in range(4):
                p = (2 + r) % 4
                q = (p + 2) % 4
                wait_loads(p)
                issue_scatter(p)
                wait_scatter(q)

                @pl.when(w0 + r + 2 < _SNFULL)
                def _():
                    issue_loads(w0 + r + 2, q)

        wait_loads(2)
        issue_scatter(2)
        wait_scatter(0)
        wait_loads(3)
        issue_scatter(3)
        wait_scatter(1)
        wait_scatter(2)
        wait_scatter(3)

        toff = ebase + _SNFULL * _SWIN
        pltpu.sync_copy(dst_hbm.at[pl.ds(toff, _STAIL)], idxt)
        pltpu.sync_copy(z_hbm.at[pl.ds(toff, _STAIL), pl.ds(c * _HW, _HW)], zbt)
        pltpu.sync_copy(zbt, acc.at[idxt], add=True)
        plsc.subcore_barrier()

        pltpu.sync_copy(acc.at[pl.ds(rbase, _NPS)],
                        o_hbm.at[pl.ds(rbase, _NPS), pl.ds(c * _HW, _HW)])

        @pl.when(s == 15)
        def _():
            pltpu.sync_copy(
                acc.at[pl.ds(16 * _NPS, _NREM)],
                o_hbm.at[pl.ds(16 * _NPS, _NREM), pl.ds(c * _HW, _HW)])

    return k(z, dst)


def _block_diag(w):
    lead = w.shape[:-3]
    out = jnp.zeros(lead + (_G, _GS, _G, _GS), w.dtype)
    wt = jnp.swapaxes(w, -1, -2)
    for g in range(_G):
        out = out.at[..., g, :, g, :].set(wt[..., g, :, :])
    return out.reshape(lead + (_W, _W))


def kernel(x, x_res, edge_index_0, edge_index_1, edge_index_2,
           edge_attr_0, edge_attr_1, edge_attr_2,
           node_deg_0, node_deg_1, node_deg_2,
           src_w, tgt_w, bond_tables, gate_w, value_w, post_w, deg_p,
           mix_pre_w, mix_gate_w, mix_value_w, mix_post_w,
           mix_sca_pre, mix_sca_post):
    eis = [edge_index_0, edge_index_1, edge_index_2]
    eas = [edge_attr_0, edge_attr_1, edge_attr_2]
    degs = [node_deg_0, node_deg_1, node_deg_2]
    srcs = [e[0] for e in eis]
    dsts = [e[1] for e in eis]
    deg2 = [d[:, None] for d in degs]

    swT = jnp.swapaxes(src_w, -1, -2)
    twT = jnp.swapaxes(tgt_w, -1, -2)
    pwT = jnp.swapaxes(post_w, -1, -2)
    gwbd = _block_diag(gate_w)
    vwbd = _block_diag(value_w)
    mix_gwbd = _block_diag(mix_gate_w)
    mix_vwbd = _block_diag(mix_value_w)
    mix_preT = mix_pre_w.T
    mix_postT = mix_post_w.T
    sp2 = mix_sca_pre[None, :]
    spo2 = mix_sca_post[None, :]

    gmat = jnp.kron(jnp.eye(_G, dtype=jnp.float32),
                    jnp.ones((_GS, 1), jnp.float32))
    gmatT = gmat.T

    x_hop = x
    x_kernel = x
    x_out = None
    for k in range(2):
        for h in range(3):
            i = k * 3 + h
            xs, xt = _node_mm(x_hop, swT[i], twT[i])
            xx = _sc_gather(xs, xt, srcs[h], dsts[h])
            z = _edge_ffn(xx, eas[h], bond_tables[i], gmat, gmatT,
                          gwbd[i], vwbd[i])
            zagg = _sc_scatter(z, dsts[h])
            x_hop, x_out = _post(zagg, pwT[i], deg2[h], deg_p[i][None, :],
                                 x_out, x_res if i == 5 else None)
        if k == 0:
            x_kernel = _mix(x_kernel, x_out, mix_preT, mix_gwbd, mix_vwbd,
                            mix_postT, sp2, spo2, gmat, gmatT)
            x_hop = x_kernel
            x_out = None
    return x_out


# speedup vs baseline: 1.5079x; 1.0005x over previous
import functools

import jax
import jax.numpy as jnp
from jax import lax
from jax.experimental import pallas as pl
from jax.experimental.pallas import tpu as pltpu
from jax.experimental.pallas import tpu_sc as plsc

_N = 10000
_E = 160000
_W = 256
_G = 8
_GS = _W // _G
_NBOND = 33
_EPS = 1e-5

_BN = 1000
_BE = 640
_PREC = lax.Precision.DEFAULT

_GWIN = 112
_PERW = _E // 32
_GNFULL = _PERW // _GWIN
_GPAIRS = _GNFULL // 2
_GTAIL = _PERW - _GNFULL * _GWIN

_SWIN = 64
_PERS = _E // 16
_SNFULL = _PERS // _SWIN
_SQUADS = (_SNFULL - 4) // 4
_STAIL = _PERS - _SNFULL * _SWIN
_HW = _W // 2
_NPS = 624
_NREM = _N - 16 * _NPS


def _cdiv(a, b):
    return (a + b - 1) // b


def _dot(a, b):
    return jnp.dot(a, b, preferred_element_type=jnp.float32, precision=_PREC)


def _gn(xx, gmat, gmatT):
    s1 = _dot(xx, gmat) * (1.0 / _GS)
    s2 = _dot(xx * xx, gmat) * (1.0 / _GS)
    mu = _dot(s1, gmatT)
    ex2 = _dot(s2, gmatT)
    var = ex2 - mu * mu
    return (xx - mu) * lax.rsqrt(var + _EPS)


def _node_mm_body(x_ref, wa_ref, wb_ref, oa_ref, ob_ref):
    x = x_ref[...]
    oa_ref[...] = _dot(x, wa_ref[...])
    ob_ref[...] = _dot(x, wb_ref[...])


def _node_mm(x, wa, wb):
    return pl.pallas_call(
        _node_mm_body,
        grid=(_cdiv(_N, _BN),),
        in_specs=[
            pl.BlockSpec((_BN, _W), lambda i: (i, 0)),
            pl.BlockSpec((_W, _W), lambda i: (0, 0)),
            pl.BlockSpec((_W, _W), lambda i: (0, 0)),
        ],
        out_specs=[pl.BlockSpec((_BN, _W), lambda i: (i, 0))] * 2,
        out_shape=[jax.ShapeDtypeStruct((_N, _W), jnp.float32)] * 2,
    )(x, wa, wb)


def _edge_ffn_body(xx_ref, attr_ref, table_ref, gmat_ref, gmatT_ref,
                   gw_ref, vw_ref, o_ref):
    xx = xx_ref[...]
    xn = _gn(xx, gmat_ref[...], gmatT_ref[...])
    a = attr_ref[...]
    iot = lax.broadcasted_iota(jnp.int32, (1, _NBOND), 1)
    cnts = jnp.zeros((a.shape[0], _NBOND), jnp.float32)
    for j in range(3):
        cnts = cnts + (a[:, j:j + 1] == iot).astype(jnp.float32)
    nz = (iot != 0).astype(jnp.float32)
    cnts = cnts * nz
    denom = jnp.maximum(jnp.sum(cnts, axis=1, keepdims=True), 1.0)
    gb = _dot(cnts, table_ref[...]) / denom
    g = jnp.maximum(_dot(xn + gb, gw_ref[...]), 0.0)
    v = _dot(xn, vw_ref[...])
    o_ref[...] = g * v


def _edge_ffn(xx, attr, table, gmat, gmatT, gwbd, vwbd):
    full = lambda r, c: pl.BlockSpec((r, c), lambda i: (0, 0))
    return pl.pallas_call(
        _edge_ffn_body,
        grid=(_cdiv(_E, _BE),),
        in_specs=[
            pl.BlockSpec((_BE, _W), lambda i: (i, 0)),
            pl.BlockSpec((_BE, 3), lambda i: (i, 0)),
            full(_NBOND, _W),
            full(_W, _G),
            full(_G, _W),
            full(_W, _W),
            full(_W, _W),
        ],
        out_specs=pl.BlockSpec((_BE, _W), lambda i: (i, 0)),
        out_shape=jax.ShapeDtypeStruct((_E, _W), jnp.float32),
    )(xx, attr, table, gmat, gmatT, gwbd, vwbd)


def _post_body(z_ref, w_ref, deg_ref, dp_ref, *rest):
    agg = _dot(z_ref[...], w_ref[...])
    scale = jnp.exp(dp_ref[...] * jnp.log(deg_ref[...]))
    hop = agg * scale
    return hop, rest


def _post_first_body(z_ref, w_ref, deg_ref, dp_ref, oh_ref):
    hop, _ = _post_body(z_ref, w_ref, deg_ref, dp_ref)
    oh_ref[...] = hop


def _post_mid_body(z_ref, w_ref, deg_ref, dp_ref, xo_ref, oh_ref, oo_ref):
    hop, _ = _post_body(z_ref, w_ref, deg_ref, dp_ref)
    oh_ref[...] = hop
    oo_ref[...] = xo_ref[...] + hop


def _post_last_body(z_ref, w_ref, deg_ref, dp_ref, xo_ref, res_ref, oo_ref):
    hop, _ = _post_body(z_ref, w_ref, deg_ref, dp_ref)
    oo_ref[...] = xo_ref[...] + hop + res_ref[...]


def _post(zagg, pwT, deg, dp, xout, res):
    row = pl.BlockSpec((_BN, _W), lambda i: (i, 0))
    specs = [
        row,
        pl.BlockSpec((_W, _W), lambda i: (0, 0)),
        pl.BlockSpec((_BN, 1), lambda i: (i, 0)),
        pl.BlockSpec((1, _W), lambda i: (0, 0)),
    ]
    args = [zagg, pwT, deg, dp]
    shp = jax.ShapeDtypeStruct((_N, _W), jnp.float32)
    if xout is None:
        out = pl.pallas_call(
            _post_first_body, grid=(_cdiv(_N, _BN),),
            in_specs=specs, out_specs=row, out_shape=shp,
            )(*args)
        return out, out
    if res is None:
        hop, xo = pl.pallas_call(
            _post_mid_body, grid=(_cdiv(_N, _BN),),
            in_specs=specs + [row], out_specs=[row, row], out_shape=[shp, shp],
            )(*args, xout)
        return hop, xo
    xo = pl.pallas_call(
        _post_last_body, grid=(_cdiv(_N, _BN),),
        in_specs=specs + [row, row], out_specs=row, out_shape=shp,
        )(*args, xout, res)
    return None, xo


def _mix_body(x_ref, res_ref, preT_ref, gw_ref, vw_ref, postT_ref,
              sp_ref, spo_ref, gmat_ref, gmatT_ref, o_ref):
    xx = jnp.exp(sp_ref[...]) * x_ref[...] + res_ref[...]
    xn = _gn(_dot(xx, preT_ref[...]), gmat_ref[...], gmatT_ref[...])
    g = jnp.maximum(_dot(xn, gw_ref[...]), 0.0)
    v = _dot(xn, vw_ref[...])
    o_ref[...] = jnp.exp(spo_ref[...]) * xx + _dot(g * v, postT_ref[...])


def _mix(x, res, preT, gwbd, vwbd, postT, sp, spo, gmat, gmatT):
    row = pl.BlockSpec((_BN, _W), lambda i: (i, 0))
    full = lambda r, c: pl.BlockSpec((r, c), lambda i: (0, 0))
    return pl.pallas_call(
        _mix_body,
        grid=(_cdiv(_N, _BN),),
        in_specs=[row, row, full(_W, _W), full(_W, _W), full(_W, _W),
                  full(_W, _W), full(1, _W), full(1, _W),
                  full(_W, _G), full(_G, _W)],
        out_specs=row,
        out_shape=jax.ShapeDtypeStruct((_N, _W), jnp.float32),
    )(x, res, preT, gwbd, vwbd, postT, sp, spo, gmat, gmatT)


def _sc_gather(xs, xt, src, dst):
    mesh = plsc.VectorSubcoreMesh(core_axis_name="c", subcore_axis_name="s")

    @functools.partial(
        pl.kernel, mesh=mesh,
        out_type=jax.ShapeDtypeStruct((_E, _W), jnp.float32),
        scratch_types=[
            pltpu.VMEM((_GWIN,), jnp.int32),
            pltpu.VMEM((_GWIN,), jnp.int32),
            pltpu.VMEM((_GWIN,), jnp.int32),
            pltpu.VMEM((_GWIN,), jnp.int32),
            pltpu.VMEM((_GWIN, _W), jnp.float32),
            pltpu.VMEM((_GWIN, _W), jnp.float32),
            pltpu.VMEM((_GWIN, _W), jnp.float32),
            pltpu.VMEM((_GWIN, _W), jnp.float32),
        ] + [pltpu.SemaphoreType.DMA] * 8)
    def k(xs_hbm, xt_hbm, src_hbm, dst_hbm, os_hbm,
          si0, si1, di0, di1, bs0, bs1, bt0, bt1,
          six0, six1, sga0, sga1, sgb0, sgb1, swa0, swa1):
        si = (si0, si1)
        di = (di0, di1)
        bs = (bs0, bs1)
        bt = (bt0, bt1)
        six = (six0, six1)
        sga = (sga0, sga1)
        sgb = (sgb0, sgb1)
        swa = (swa0, swa1)
        wid = lax.axis_index("s") * 2 + lax.axis_index("c")
        base = wid * _PERW

        def issue_idx(w, p):
            pltpu.async_copy(src_hbm.at[pl.ds(base + w * _GWIN, _GWIN)],
                             si[p], six[p])
            pltpu.async_copy(dst_hbm.at[pl.ds(base + w * _GWIN, _GWIN)],
                             di[p], six[p])

        def wait_idx(p):
            pltpu.make_async_copy(src_hbm.at[pl.ds(0, _GWIN)],
                                  si[p], six[p]).wait()
            pltpu.make_async_copy(dst_hbm.at[pl.ds(0, _GWIN)],
                                  di[p], six[p]).wait()

        def issue_gather(p):
            pltpu.async_copy(xs_hbm.at[si[p]], bs[p], sga[p])
            pltpu.async_copy(xt_hbm.at[di[p]], bt[p], sgb[p])

        def wait_gather(p):
            pltpu.make_async_copy(xs_hbm.at[si[p]], bs[p], sga[p]).wait()
            pltpu.make_async_copy(xt_hbm.at[di[p]], bt[p], sgb[p]).wait()

        def add_rows(p, nrows):
            a = bs[p]
            b = bt[p]

            @pl.loop(0, nrows)
            def _(r):
                for j in range(_W // 16):
                    sl = pl.ds(j * 16, 16)
                    a[r, sl] = a[r, sl] + b[r, sl]

        def issue_wb(w, p):
            pltpu.async_copy(bs[p], os_hbm.at[pl.ds(base + w * _GWIN, _GWIN)],
                             swa[p])

        def wait_wb(p):
            pltpu.make_async_copy(bs[p], os_hbm.at[pl.ds(0, _GWIN)],
                                  swa[p]).wait()

        issue_idx(0, 0)
        issue_idx(1, 1)
        wait_idx(0)
        issue_gather(0)
        wait_idx(1)
        issue_gather(1)
        wait_gather(0)
        add_rows(0, _GWIN)
        issue_wb(0, 0)
        issue_idx(2, 0)

        @pl.loop(1, _GPAIRS)
        def _(j):
            wa = 2 * j
            wait_wb(0)
            wait_idx(0)
            issue_gather(0)
            wait_gather(1)
            add_rows(1, _GWIN)
            issue_wb(wa - 1, 1)
            issue_idx(wa + 1, 1)
            wait_wb(1)
            wait_idx(1)
            issue_gather(1)
            wait_gather(0)
            add_rows(0, _GWIN)
            issue_wb(wa, 0)

            @pl.when(j < _GPAIRS - 1)
            def _():
                issue_idx(wa + 2, 0)

        wait_gather(1)
        add_rows(1, _GWIN)
        issue_wb(_GNFULL - 1, 1)

        toff = base + _GNFULL * _GWIN
        wait_wb(0)
        h1 = pltpu.async_copy(src_hbm.at[pl.ds(toff, _GTAIL)],
                              si0.at[pl.ds(0, _GTAIL)], six0)
        h2 = pltpu.async_copy(dst_hbm.at[pl.ds(toff, _GTAIL)],
                              di0.at[pl.ds(0, _GTAIL)], six0)
        h1.wait()
        h2.wait()
        g1 = pltpu.async_copy(xs_hbm.at[si0.at[pl.ds(0, _GTAIL)]],
                              bs0.at[pl.ds(0, _GTAIL)], sga0)
        g2 = pltpu.async_copy(xt_hbm.at[di0.at[pl.ds(0, _GTAIL)]],
                              bt0.at[pl.ds(0, _GTAIL)], sgb0)
        g1.wait()
        g2.wait()
        add_rows(0, _GTAIL)
        o1 = pltpu.async_copy(bs0.at[pl.ds(0, _GTAIL)],
                              os_hbm.at[pl.ds(toff, _GTAIL)], swa0)
        o1.wait()
        wait_wb(1)

    return k(xs, xt, src, dst)


def _sc_scatter(z, dst):
    mesh = plsc.VectorSubcoreMesh(core_axis_name="c", subcore_axis_name="s")

    @functools.partial(
        pl.kernel, mesh=mesh,
        out_type=jax.ShapeDtypeStruct((_N, _W), jnp.float32),
        scratch_types=[
            pltpu.VMEM((_SWIN,), jnp.int32),
            pltpu.VMEM((_SWIN,), jnp.int32),
            pltpu.VMEM((_SWIN,), jnp.int32),
            pltpu.VMEM((_SWIN,), jnp.int32),
            pltpu.VMEM((_STAIL,), jnp.int32),
            pltpu.VMEM((_SWIN, _HW), jnp.float32),
            pltpu.VMEM((_SWIN, _HW), jnp.float32),
            pltpu.VMEM((_SWIN, _HW), jnp.float32),
            pltpu.VMEM((_SWIN, _HW), jnp.float32),
            pltpu.VMEM((_STAIL, _HW), jnp.float32),
            pltpu.VMEM_SHARED((_N, _HW), jnp.float32),
        ] + [pltpu.SemaphoreType.DMA] * 8)
    def k(z_hbm, dst_hbm, o_hbm, idx0, idx1, idx2, idx3, idxt,
          zb0, zb1, zb2, zb3, zbt, acc,
          sl0, sl1, sl2, sl3, ss0, ss1, ss2, ss3):
        idx = (idx0, idx1, idx2, idx3)
        zb = (zb0, zb1, zb2, zb3)
        sl = (sl0, sl1, sl2, sl3)
        ss = (ss0, ss1, ss2, ss3)
        c = lax.axis_index("c")
        s = lax.axis_index("s")

        @pl.loop(0, _SWIN)
        def _(r):
            for j in range(_HW // 16):
                zb0[r, pl.ds(j * 16, 16)] = jnp.zeros((16,), jnp.float32)

        rbase = s * _NPS
        zchunks = [(o, _SWIN) for o in range(0, _NPS - _SWIN + 1, _SWIN)]
        if _NPS % _SWIN:
            zchunks.append((_NPS - _NPS % _SWIN, _NPS % _SWIN))
        for (o, n) in zchunks:
            pltpu.sync_copy(zb0.at[pl.ds(0, n)], acc.at[pl.ds(rbase + o, n)])

        @pl.when(s == 15)
        def _():
            pltpu.sync_copy(zb0.at[pl.ds(0, _NREM)],
                            acc.at[pl.ds(16 * _NPS, _NREM)])
        plsc.subcore_barrier()

        ebase = s * _PERS

        def issue_loads(w, p):
            off = ebase + w * _SWIN
            pltpu.async_copy(dst_hbm.at[pl.ds(off, _SWIN)], idx[p], sl[p])
            pltpu.async_copy(z_hbm.at[pl.ds(off, _SWIN), pl.ds(c * _HW, _HW)],
                             zb[p], sl[p])

        def wait_loads(p):
            pltpu.make_async_copy(dst_hbm.at[pl.ds(0, _SWIN)],
                                  idx[p], sl[p]).wait()
            pltpu.make_async_copy(z_hbm.at[pl.ds(0, _SWIN), pl.ds(0, _HW)],
                                  zb[p], sl[p]).wait()

        def issue_scatter(p):
            pltpu.async_copy(zb[p], acc.at[idx[p]], ss[p], add=True)

        def wait_scatter(p):
            pltpu.make_async_copy(zb[p], acc.at[idx[p]], ss[p]).wait()

        issue_loads(0, 0)
        issue_loads(1, 1)
        wait_loads(0)
        issue_scatter(0)
        issue_loads(2, 2)
        wait_loads(1)
        issue_scatter(1)
        issue_loads(3, 3)

        @pl.loop(0, _SQUADS)
        def _(j):
            w0 = 2 + 4 * j
            for r in range(4):
                p = (2 + r) % 4
                q = (p + 2) % 4
                wait_loads(p)
                issue_scatter(p)
                wait_scatter(q)

                @pl.when(w0 + r + 2 < _SNFULL)
                def _():
                    issue_loads(w0 + r + 2, q)

        wait_loads(2)
        issue_scatter(2)
        wait_scatter(0)
        wait_loads(3)
        issue_scatter(3)
        wait_scatter(1)
        wait_scatter(2)
        wait_scatter(3)

        toff = ebase + _SNFULL * _SWIN
        pltpu.sync_copy(dst_hbm.at[pl.ds(toff, _STAIL)], idxt)
        pltpu.sync_copy(z_hbm.at[pl.ds(toff, _STAIL), pl.ds(c * _HW, _HW)], zbt)
        pltpu.sync_copy(zbt, acc.at[idxt], add=True)
        plsc.subcore_barrier()

        pltpu.sync_copy(acc.at[pl.ds(rbase, _NPS)],
                        o_hbm.at[pl.ds(rbase, _NPS), pl.ds(c * _HW, _HW)])

        @pl.when(s == 15)
        def _():
            pltpu.sync_copy(
                acc.at[pl.ds(16 * _NPS, _NREM)],
                o_hbm.at[pl.ds(16 * _NPS, _NREM), pl.ds(c * _HW, _HW)])

    return k(z, dst)


def _block_diag(w):
    lead = w.shape[:-3]
    out = jnp.zeros(lead + (_G, _GS, _G, _GS), w.dtype)
    wt = jnp.swapaxes(w, -1, -2)
    for g in range(_G):
        out = out.at[..., g, :, g, :].set(wt[..., g, :, :])
    return out.reshape(lead + (_W, _W))


def kernel(x, x_res, edge_index_0, edge_index_1, edge_index_2,
           edge_attr_0, edge_attr_1, edge_attr_2,
           node_deg_0, node_deg_1, node_deg_2,
           src_w, tgt_w, bond_tables, gate_w, value_w, post_w, deg_p,
           mix_pre_w, mix_gate_w, mix_value_w, mix_post_w,
           mix_sca_pre, mix_sca_post):
    eis = [edge_index_0, edge_index_1, edge_index_2]
    eas = [edge_attr_0, edge_attr_1, edge_attr_2]
    degs = [node_deg_0, node_deg_1, node_deg_2]
    srcs = [e[0] for e in eis]
    dsts = [e[1] for e in eis]
    deg2 = [d[:, None] for d in degs]

    swT = jnp.swapaxes(src_w, -1, -2)
    twT = jnp.swapaxes(tgt_w, -1, -2)
    pwT = jnp.swapaxes(post_w, -1, -2)
    gwbd = _block_diag(gate_w)
    vwbd = _block_diag(value_w)
    mix_gwbd = _block_diag(mix_gate_w)
    mix_vwbd = _block_diag(mix_value_w)
    mix_preT = mix_pre_w.T
    mix_postT = mix_post_w.T
    sp2 = mix_sca_pre[None, :]
    spo2 = mix_sca_post[None, :]

    gmat = jnp.kron(jnp.eye(_G, dtype=jnp.float32),
                    jnp.ones((_GS, 1), jnp.float32))
    gmatT = gmat.T

    x_hop = x
    x_kernel = x
    x_out = None
    for k in range(2):
        for h in range(3):
            i = k * 3 + h
            xs, xt = _node_mm(x_hop, swT[i], twT[i])
            xx = _sc_gather(xs, xt, srcs[h], dsts[h])
            z = _edge_ffn(xx, eas[h], bond_tables[i], gmat, gmatT,
                          gwbd[i], vwbd[i])
            zagg = _sc_scatter(z, dsts[h])
            x_hop, x_out = _post(zagg, pwT[i], deg2[h], deg_p[i][None, :],
                                 x_out, x_res if i == 5 else None)
        if k == 0:
            x_kernel = _mix(x_kernel, x_out, mix_preT, mix_gwbd, mix_vwbd,
                            mix_postT, sp2, spo2, gmat, gmatT)
            x_hop = x_kernel
            x_out = None
    return x_out
